# baseline (speedup 1.0000x reference)
import math

import jax
import jax.numpy as jnp
from jax import lax
from jax.experimental import pallas as pl
from jax.experimental.pallas import tpu as pltpu
from jax.experimental.pallas import tpu_sc as plsc

N_NODES = 10000
N_EDGES = 320000
D_FEAT = 128
_FACTOR = 1.0 / math.sqrt(32.0)

_NC = 2
_NS = 16
_EPW = N_EDGES // (_NC * _NS)
_CHUNK = 80
_NITER = _EPW // _CHUNK
_ROWS_PER_SUB = 624
_TAIL_ROW = _NS * _ROWS_PER_SUB
_TAIL = N_NODES - _TAIL_ROW
_NBUF = 3
_NMAIN = 41
_ZROWS = 16


def _gather(feat_hbm, rows_v, gsem, ebase, j, b):
    return pltpu.async_copy(
        feat_hbm.at[pl.ds(ebase + j * _CHUNK, _CHUNK)], rows_v.at[b],
        gsem.at[b],
    )


def _sc_body(feat_hbm, dst_hbm, part_hbm, idx_v, rows_v, zero_v, acc_sh,
             gsem, ssem, zsem, isem):
    c = lax.axis_index("c")
    s = lax.axis_index("s")
    wid = c * _NS + s
    ebase = wid * _EPW

    for b in range(_NBUF):
        _gather(feat_hbm, rows_v, gsem, ebase, b, b)
    idx_cp = pltpu.async_copy(dst_hbm.at[0].at[wid], idx_v, isem)

    zvec = jnp.zeros((16,), jnp.float32)

    @pl.loop(0, _ZROWS)
    def _(i):
        @pl.loop(0, D_FEAT, step=16)
        def _(j):
            zero_v.at[i, pl.ds(j, 16)][...] = zvec

    base_row = s * _ROWS_PER_SUB

    @pl.loop(0, _ROWS_PER_SUB, step=_ZROWS)
    def _(r):
        pltpu.async_copy(
            zero_v, acc_sh.at[pl.ds(base_row + r, _ZROWS)], zsem
        )

    @pl.when(s == _NS - 1)
    def _():
        pltpu.async_copy(
            zero_v.at[pl.ds(0, _TAIL)], acc_sh.at[pl.ds(_TAIL_ROW, _TAIL)],
            zsem,
        )

    @pl.loop(0, _ROWS_PER_SUB, step=_ZROWS)
    def _(r):
        pltpu.make_async_copy(
            zero_v, acc_sh.at[pl.ds(base_row + r, _ZROWS)], zsem
        ).wait()

    @pl.when(s == _NS - 1)
    def _():
        pltpu.make_async_copy(
            zero_v.at[pl.ds(0, _TAIL)], acc_sh.at[pl.ds(_TAIL_ROW, _TAIL)],
            zsem,
        ).wait()

    idx_cp.wait()
    plsc.subcore_barrier()


    @pl.loop(0, _NMAIN)
    def _(g):
        for b in range(_NBUF):
            j = g * _NBUF + b
            pltpu.make_async_copy(
                feat_hbm.at[pl.ds(ebase + j * _CHUNK, _CHUNK)],
                rows_v.at[b],
                gsem.at[b],
            ).wait()
            pltpu.async_copy(
                rows_v.at[b], acc_sh.at[idx_v.at[j]], ssem.at[b], add=True
            )
        for b in range(_NBUF):
            j = g * _NBUF + b
            pltpu.make_async_copy(
                rows_v.at[b], acc_sh.at[idx_v.at[j]], ssem.at[b]
            ).wait()

            @pl.when(j + _NBUF < _NITER)
            def _():
                _gather(feat_hbm, rows_v, gsem, ebase, j + _NBUF, b)

    tail = []
    for b, j in [(0, 123), (1, 124)]:
        pltpu.make_async_copy(
            feat_hbm.at[pl.ds(ebase + j * _CHUNK, _CHUNK)],
            rows_v.at[b],
            gsem.at[b],
        ).wait()
        tail.append(
            pltpu.async_copy(
                rows_v.at[b], acc_sh.at[idx_v.at[j]], ssem.at[b], add=True
            )
        )
    for d in tail:
        d.wait()

    plsc.subcore_barrier()

    out_row = c * N_NODES + base_row
    pltpu.sync_copy(
        acc_sh.at[pl.ds(base_row, _ROWS_PER_SUB)],
        part_hbm.at[pl.ds(out_row, _ROWS_PER_SUB)],
    )

    @pl.when(s == _NS - 1)
    def _():
        pltpu.sync_copy(
            acc_sh.at[pl.ds(_TAIL_ROW, _TAIL)],
            part_hbm.at[pl.ds(c * N_NODES + _TAIL_ROW, _TAIL)],
        )


def _combine_body(p_ref, o_ref):
    o_ref[...] = (p_ref[0] + p_ref[1]) * _FACTOR


def kernel(edge_feat, edge_index, pos):
    edge_dst = edge_index.reshape(2, _NC * _NS, _NITER, _CHUNK)

    mesh = plsc.VectorSubcoreMesh(core_axis_name="c", subcore_axis_name="s")
    sc_scatter = pl.kernel(
        _sc_body,
        out_type=jax.ShapeDtypeStruct((_NC * N_NODES, D_FEAT), jnp.float32),
        mesh=mesh,
        scratch_types=[
            pltpu.VMEM((_NITER, _CHUNK), jnp.int32),
            pltpu.VMEM((_NBUF, _CHUNK, D_FEAT), jnp.float32),
            pltpu.VMEM((_ZROWS, D_FEAT), jnp.float32),
            pltpu.VMEM_SHARED((N_NODES, D_FEAT), jnp.float32),
            pltpu.SemaphoreType.DMA((_NBUF,)),
            pltpu.SemaphoreType.DMA((_NBUF,)),
            pltpu.SemaphoreType.DMA,
            pltpu.SemaphoreType.DMA,
        ],
    )
    partials = sc_scatter(edge_feat, edge_dst)

    rows_blk = 1000
    out = pl.pallas_call(
        _combine_body,
        out_shape=jax.ShapeDtypeStruct((N_NODES, D_FEAT), jnp.float32),
        grid=(N_NODES // rows_blk,),
        in_specs=[
            pl.BlockSpec((_NC, rows_blk, D_FEAT), lambda i: (0, i, 0)),
        ],
        out_specs=pl.BlockSpec((rows_blk, D_FEAT), lambda i: (i, 0)),
    )(partials.reshape(_NC, N_NODES, D_FEAT))
    return out


# speedup vs baseline: 1.0103x; 1.0103x over previous
import math

import jax
import jax.numpy as jnp
from jax import lax
from jax.experimental import pallas as pl
from jax.experimental.pallas import tpu as pltpu
from jax.experimental.pallas import tpu_sc as plsc

N_NODES = 10000
N_EDGES = 320000
D_FEAT = 128
_FACTOR = 1.0 / math.sqrt(32.0)

_NC = 2
_NS = 16
_EPW = N_EDGES // (_NC * _NS)
_CHUNK = 80
_NITER = _EPW // _CHUNK
_ROWS_PER_SUB = 624
_TAIL_ROW = _NS * _ROWS_PER_SUB
_TAIL = N_NODES - _TAIL_ROW
_NBUF = 3
_NMAIN = 41
_ZROWS = 16


def _gather(feat_hbm, rows_v, gsem, ebase, j, b):
    return pltpu.async_copy(
        feat_hbm.at[pl.ds(ebase + j * _CHUNK, _CHUNK)], rows_v.at[b],
        gsem.at[b],
    )


def _sc_body(feat_hbm, dst_hbm, part_hbm, idx_v, rows_v, zero_v, acc_sh,
             gsem, ssem, zsem, isem):
    c = lax.axis_index("c")
    s = lax.axis_index("s")
    wid = c * _NS + s
    ebase = wid * _EPW

    for b in range(_NBUF):
        _gather(feat_hbm, rows_v, gsem, ebase, b, b)
    idx_cp = pltpu.async_copy(dst_hbm.at[0].at[wid], idx_v, isem)

    zvec = jnp.zeros((16,), jnp.float32)

    @pl.loop(0, _ZROWS)
    def _(i):
        @pl.loop(0, D_FEAT, step=16)
        def _(j):
            zero_v.at[i, pl.ds(j, 16)][...] = zvec

    base_row = s * _ROWS_PER_SUB

    @pl.loop(0, _ROWS_PER_SUB, step=_ZROWS)
    def _(r):
        pltpu.async_copy(
            zero_v, acc_sh.at[pl.ds(base_row + r, _ZROWS)], zsem
        )

    @pl.when(s == _NS - 1)
    def _():
        pltpu.async_copy(
            zero_v.at[pl.ds(0, _TAIL)], acc_sh.at[pl.ds(_TAIL_ROW, _TAIL)],
            zsem,
        )

    @pl.loop(0, _ROWS_PER_SUB, step=_ZROWS)
    def _(r):
        pltpu.make_async_copy(
            zero_v, acc_sh.at[pl.ds(base_row + r, _ZROWS)], zsem
        ).wait()

    @pl.when(s == _NS - 1)
    def _():
        pltpu.make_async_copy(
            zero_v.at[pl.ds(0, _TAIL)], acc_sh.at[pl.ds(_TAIL_ROW, _TAIL)],
            zsem,
        ).wait()

    idx_cp.wait()
    plsc.subcore_barrier()


    @pl.loop(0, _NMAIN)
    def _(g):
        for b in range(_NBUF):
            j = g * _NBUF + b
            pltpu.make_async_copy(
                feat_hbm.at[pl.ds(ebase + j * _CHUNK, _CHUNK)],
                rows_v.at[b],
                gsem.at[b],
            ).wait()
            pltpu.async_copy(
                rows_v.at[b], acc_sh.at[idx_v.at[j]], ssem.at[b], add=True
            )
        for b in range(_NBUF):
            j = g * _NBUF + b
            pltpu.make_async_copy(
                rows_v.at[b], acc_sh.at[idx_v.at[j]], ssem.at[b]
            ).wait()

            @pl.when(j + _NBUF < _NITER)
            def _():
                _gather(feat_hbm, rows_v, gsem, ebase, j + _NBUF, b)

    tail = []
    for b, j in [(0, 123), (1, 124)]:
        pltpu.make_async_copy(
            feat_hbm.at[pl.ds(ebase + j * _CHUNK, _CHUNK)],
            rows_v.at[b],
            gsem.at[b],
        ).wait()
        tail.append(
            pltpu.async_copy(
                rows_v.at[b], acc_sh.at[idx_v.at[j]], ssem.at[b], add=True
            )
        )
    for d in tail:
        d.wait()

    plsc.subcore_barrier()

    out_row = c * N_NODES + base_row
    pltpu.sync_copy(
        acc_sh.at[pl.ds(base_row, _ROWS_PER_SUB)],
        part_hbm.at[pl.ds(out_row, _ROWS_PER_SUB)],
    )

    @pl.when(s == _NS - 1)
    def _():
        pltpu.sync_copy(
            acc_sh.at[pl.ds(_TAIL_ROW, _TAIL)],
            part_hbm.at[pl.ds(c * N_NODES + _TAIL_ROW, _TAIL)],
        )


def _combine_body(p_ref, o_ref):
    o_ref[...] = (p_ref[0] + p_ref[1]) * _FACTOR


def kernel(edge_feat, edge_index, pos):
    edge_dst = edge_index.reshape(2, _NC * _NS, _NITER, _CHUNK)

    mesh = plsc.VectorSubcoreMesh(core_axis_name="c", subcore_axis_name="s")
    sc_scatter = pl.kernel(
        _sc_body,
        out_type=jax.ShapeDtypeStruct((_NC * N_NODES, D_FEAT), jnp.float32),
        mesh=mesh,
        scratch_types=[
            pltpu.VMEM((_NITER, _CHUNK), jnp.int32),
            pltpu.VMEM((_NBUF, _CHUNK, D_FEAT), jnp.float32),
            pltpu.VMEM((_ZROWS, D_FEAT), jnp.float32),
            pltpu.VMEM_SHARED((N_NODES, D_FEAT), jnp.float32),
            pltpu.SemaphoreType.DMA((_NBUF,)),
            pltpu.SemaphoreType.DMA((_NBUF,)),
            pltpu.SemaphoreType.DMA,
            pltpu.SemaphoreType.DMA,
        ],
    )
    partials = sc_scatter(edge_feat, edge_dst)

    rows_blk = 2000
    out = pl.pallas_call(
        _combine_body,
        out_shape=jax.ShapeDtypeStruct((N_NODES, D_FEAT), jnp.float32),
        grid=(N_NODES // rows_blk,),
        in_specs=[
            pl.BlockSpec((_NC, rows_blk, D_FEAT), lambda i: (0, i, 0)),
        ],
        out_specs=pl.BlockSpec((rows_blk, D_FEAT), lambda i: (i, 0)),
    )(partials.reshape(_NC, N_NODES, D_FEAT))
    return out
